# initial kernel scaffold (unmeasured)
import jax
import jax.numpy as jnp
from jax import lax
from jax.experimental import pallas as pl
from jax.experimental.pallas import tpu as pltpu


def kernel(
    x,
):
    def body(*refs):
        pass

    out_shape = jax.ShapeDtypeStruct(..., jnp.float32)
    return pl.pallas_call(body, out_shape=out_shape)(...)


# baseline (device time: 13396 ns/iter reference)
import jax
import jax.numpy as jnp
from jax import lax
from jax.experimental import pallas as pl
from jax.experimental.pallas import tpu as pltpu

N_DEV = 4


def kernel(x):
    _, m, n = x.shape

    def body(x_ref, out_ref, recv_buf, send_sems, recv_sems):
        my = lax.axis_index("i")
        p1 = my ^ 1
        p2 = 3 - my

        barrier_sem = pltpu.get_barrier_semaphore()
        for p in (p1, p2):
            pl.semaphore_signal(
                barrier_sem, inc=1,
                device_id=(p,), device_id_type=pl.DeviceIdType.MESH,
            )
        pl.semaphore_wait(barrier_sem, 2)

        out_ref[:, :] = x_ref[0, :, :]

        rdma1 = pltpu.make_async_remote_copy(
            src_ref=out_ref,
            dst_ref=recv_buf.at[0],
            send_sem=send_sems.at[0],
            recv_sem=recv_sems.at[0],
            device_id=(p1,),
            device_id_type=pl.DeviceIdType.MESH,
        )
        rdma1.start()
        rdma1.wait()
        out_ref[:, :] = out_ref[:, :] + recv_buf[0, :, :]

        rdma2 = pltpu.make_async_remote_copy(
            src_ref=out_ref,
            dst_ref=recv_buf.at[1],
            send_sem=send_sems.at[1],
            recv_sem=recv_sems.at[1],
            device_id=(p2,),
            device_id_type=pl.DeviceIdType.MESH,
        )
        rdma2.start()
        rdma2.wait()
        out_ref[:, :] = out_ref[:, :] + recv_buf[1, :, :]

    return pl.pallas_call(
        body,
        out_shape=jax.ShapeDtypeStruct((m, n), x.dtype),
        in_specs=[pl.BlockSpec(memory_space=pltpu.VMEM)],
        out_specs=pl.BlockSpec(memory_space=pltpu.VMEM),
        scratch_shapes=[
            pltpu.VMEM((2, m, n), x.dtype),
            pltpu.SemaphoreType.DMA((2,)),
            pltpu.SemaphoreType.DMA((2,)),
        ],
        compiler_params=pltpu.CompilerParams(collective_id=0),
    )(x)


# device time: 11339 ns/iter; 1.1814x vs baseline; 1.1814x over previous
import jax
import jax.numpy as jnp
from jax import lax
from jax.experimental import pallas as pl
from jax.experimental.pallas import tpu as pltpu

N_DEV = 4
C = 4


def kernel(x):
    _, m, n = x.shape
    mc = m // C

    def body(x_ref, out_ref, recv1, recv2,
             send1, rsem1, send2, rsem2):
        my = lax.axis_index("i")
        p1 = my ^ 1
        p2 = 3 - my

        out_ref[:, :] = x_ref[0, :, :]

        barrier_sem = pltpu.get_barrier_semaphore()
        for p in (p1, p2):
            pl.semaphore_signal(
                barrier_sem, inc=1,
                device_id=(p,), device_id_type=pl.DeviceIdType.MESH,
            )
        pl.semaphore_wait(barrier_sem, 2)

        def rows(c):
            return pl.ds(c * mc, mc)

        rdma1 = []
        for c in range(C):
            r = pltpu.make_async_remote_copy(
                src_ref=out_ref.at[rows(c), :],
                dst_ref=recv1.at[rows(c), :],
                send_sem=send1.at[c],
                recv_sem=rsem1.at[c],
                device_id=(p1,),
                device_id_type=pl.DeviceIdType.MESH,
            )
            r.start()
            rdma1.append(r)

        rdma2 = []
        for c in range(C):
            rdma1[c].wait()
            out_ref[rows(c), :] = out_ref[rows(c), :] + recv1[rows(c), :]
            r = pltpu.make_async_remote_copy(
                src_ref=out_ref.at[rows(c), :],
                dst_ref=recv2.at[rows(c), :],
                send_sem=send2.at[c],
                recv_sem=rsem2.at[c],
                device_id=(p2,),
                device_id_type=pl.DeviceIdType.MESH,
            )
            r.start()
            rdma2.append(r)

        for c in range(C):
            rdma2[c].wait()
            out_ref[rows(c), :] = out_ref[rows(c), :] + recv2[rows(c), :]

    return pl.pallas_call(
        body,
        out_shape=jax.ShapeDtypeStruct((m, n), x.dtype),
        in_specs=[pl.BlockSpec(memory_space=pltpu.VMEM)],
        out_specs=pl.BlockSpec(memory_space=pltpu.VMEM),
        scratch_shapes=[
            pltpu.VMEM((m, n), x.dtype),
            pltpu.VMEM((m, n), x.dtype),
            pltpu.SemaphoreType.DMA((C,)),
            pltpu.SemaphoreType.DMA((C,)),
            pltpu.SemaphoreType.DMA((C,)),
            pltpu.SemaphoreType.DMA((C,)),
        ],
        compiler_params=pltpu.CompilerParams(collective_id=0),
    )(x)


# device time: 10005 ns/iter; 1.3389x vs baseline; 1.1333x over previous
import jax
import jax.numpy as jnp
from jax import lax
from jax.experimental import pallas as pl
from jax.experimental.pallas import tpu as pltpu

N_DEV = 4
C = 2


def kernel(x):
    _, m, n = x.shape
    half = m // 2
    sub = half // C
    M = 2 * C

    def body(x_ref, out_ref, recv1, recv2,
             send1, rsem1, send2, rsem2):
        my = lax.axis_index("i")
        p1 = my ^ 1
        p2 = 3 - my

        out_ref[:, :] = x_ref[0, :, :]

        barrier_sem = pltpu.get_barrier_semaphore()
        for p in (p1, p2):
            pl.semaphore_signal(
                barrier_sem, inc=1,
                device_id=(p,), device_id_type=pl.DeviceIdType.MESH,
            )
        pl.semaphore_wait(barrier_sem, 2)

        def rows(j):
            h, c = j % 2, j // 2
            return pl.ds(h * half + c * sub, sub)

        def partners(j):
            return (p1, p2) if j % 2 == 0 else (p2, p1)

        rdma1 = []
        for j in range(M):
            r = pltpu.make_async_remote_copy(
                src_ref=out_ref.at[rows(j), :],
                dst_ref=recv1.at[rows(j), :],
                send_sem=send1.at[j],
                recv_sem=rsem1.at[j],
                device_id=(partners(j)[0],),
                device_id_type=pl.DeviceIdType.MESH,
            )
            r.start()
            rdma1.append(r)

        rdma2 = []
        for j in range(M):
            rdma1[j].wait()
            out_ref[rows(j), :] = out_ref[rows(j), :] + recv1[rows(j), :]
            r = pltpu.make_async_remote_copy(
                src_ref=out_ref.at[rows(j), :],
                dst_ref=recv2.at[rows(j), :],
                send_sem=send2.at[j],
                recv_sem=rsem2.at[j],
                device_id=(partners(j)[1],),
                device_id_type=pl.DeviceIdType.MESH,
            )
            r.start()
            rdma2.append(r)

        for j in range(M):
            rdma2[j].wait()
            out_ref[rows(j), :] = out_ref[rows(j), :] + recv2[rows(j), :]

    return pl.pallas_call(
        body,
        out_shape=jax.ShapeDtypeStruct((m, n), x.dtype),
        in_specs=[pl.BlockSpec(memory_space=pltpu.VMEM)],
        out_specs=pl.BlockSpec(memory_space=pltpu.VMEM),
        scratch_shapes=[
            pltpu.VMEM((m, n), x.dtype),
            pltpu.VMEM((m, n), x.dtype),
            pltpu.SemaphoreType.DMA((M,)),
            pltpu.SemaphoreType.DMA((M,)),
            pltpu.SemaphoreType.DMA((M,)),
            pltpu.SemaphoreType.DMA((M,)),
        ],
        compiler_params=pltpu.CompilerParams(collective_id=0),
    )(x)


# device time: 9987 ns/iter; 1.3413x vs baseline; 1.0018x over previous
import jax
import jax.numpy as jnp
from jax import lax
from jax.experimental import pallas as pl
from jax.experimental.pallas import tpu as pltpu

N_DEV = 4
C = 2


def kernel(x):
    _, m, n = x.shape
    half = m // 2
    sub = half // C
    M = 2 * C

    def body(x_ref, out_ref, recv1, recv2,
             send1, rsem1, send2, rsem2):
        my = lax.axis_index("i")
        p1 = my ^ 1
        p2 = 3 - my

        barrier_sem = pltpu.get_barrier_semaphore()
        for p in (p1, p2):
            pl.semaphore_signal(
                barrier_sem, inc=1,
                device_id=(p,), device_id_type=pl.DeviceIdType.MESH,
            )
        pl.semaphore_wait(barrier_sem, 2)

        def rows(j):
            h, c = j % 2, j // 2
            return pl.ds(h * half + c * sub, sub)

        def partners(j):
            return (p1, p2) if j % 2 == 0 else (p2, p1)

        rdma1 = []
        for j in range(M):
            r = pltpu.make_async_remote_copy(
                src_ref=x_ref.at[0, rows(j), :],
                dst_ref=recv1.at[rows(j), :],
                send_sem=send1.at[j],
                recv_sem=rsem1.at[j],
                device_id=(partners(j)[0],),
                device_id_type=pl.DeviceIdType.MESH,
            )
            r.start()
            rdma1.append(r)

        rdma2 = []
        for j in range(M):
            rdma1[j].wait_recv()
            out_ref[rows(j), :] = x_ref[0, rows(j), :] + recv1[rows(j), :]
            r = pltpu.make_async_remote_copy(
                src_ref=out_ref.at[rows(j), :],
                dst_ref=recv2.at[rows(j), :],
                send_sem=send2.at[j],
                recv_sem=rsem2.at[j],
                device_id=(partners(j)[1],),
                device_id_type=pl.DeviceIdType.MESH,
            )
            r.start()
            rdma2.append(r)

        for j in range(M):
            rdma2[j].wait()
            out_ref[rows(j), :] = out_ref[rows(j), :] + recv2[rows(j), :]

        for j in range(M):
            rdma1[j].wait_send()

    return pl.pallas_call(
        body,
        out_shape=jax.ShapeDtypeStruct((m, n), x.dtype),
        in_specs=[pl.BlockSpec(memory_space=pltpu.VMEM)],
        out_specs=pl.BlockSpec(memory_space=pltpu.VMEM),
        scratch_shapes=[
            pltpu.VMEM((m, n), x.dtype),
            pltpu.VMEM((m, n), x.dtype),
            pltpu.SemaphoreType.DMA((M,)),
            pltpu.SemaphoreType.DMA((M,)),
            pltpu.SemaphoreType.DMA((M,)),
            pltpu.SemaphoreType.DMA((M,)),
        ],
        compiler_params=pltpu.CompilerParams(collective_id=0),
    )(x)


# device time: 9811 ns/iter; 1.3654x vs baseline; 1.0179x over previous
import jax
import jax.numpy as jnp
from jax import lax
from jax.experimental import pallas as pl
from jax.experimental.pallas import tpu as pltpu

N_DEV = 4
C = 4


def kernel(x):
    _, m, n = x.shape
    half = m // 2
    sub = half // C
    M = 2 * C

    def body(x_ref, out_ref, recv1, recv2,
             send1, rsem1, send2, rsem2):
        my = lax.axis_index("i")
        p1 = my ^ 1
        p2 = 3 - my

        barrier_sem = pltpu.get_barrier_semaphore()
        for p in (p1, p2):
            pl.semaphore_signal(
                barrier_sem, inc=1,
                device_id=(p,), device_id_type=pl.DeviceIdType.MESH,
            )
        pl.semaphore_wait(barrier_sem, 2)

        def rows(j):
            h, c = j % 2, j // 2
            return pl.ds(h * half + c * sub, sub)

        def partners(j):
            return (p1, p2) if j % 2 == 0 else (p2, p1)

        rdma1 = []
        for j in range(M):
            r = pltpu.make_async_remote_copy(
                src_ref=x_ref.at[0, rows(j), :],
                dst_ref=recv1.at[rows(j), :],
                send_sem=send1.at[j],
                recv_sem=rsem1.at[j],
                device_id=(partners(j)[0],),
                device_id_type=pl.DeviceIdType.MESH,
            )
            r.start()
            rdma1.append(r)

        rdma2 = []
        for j in range(M):
            rdma1[j].wait_recv()
            out_ref[rows(j), :] = x_ref[0, rows(j), :] + recv1[rows(j), :]
            r = pltpu.make_async_remote_copy(
                src_ref=out_ref.at[rows(j), :],
                dst_ref=recv2.at[rows(j), :],
                send_sem=send2.at[j],
                recv_sem=rsem2.at[j],
                device_id=(partners(j)[1],),
                device_id_type=pl.DeviceIdType.MESH,
            )
            r.start()
            rdma2.append(r)

        for j in range(M):
            rdma2[j].wait()
            out_ref[rows(j), :] = out_ref[rows(j), :] + recv2[rows(j), :]

        for j in range(M):
            rdma1[j].wait_send()

    return pl.pallas_call(
        body,
        out_shape=jax.ShapeDtypeStruct((m, n), x.dtype),
        in_specs=[pl.BlockSpec(memory_space=pltpu.VMEM)],
        out_specs=pl.BlockSpec(memory_space=pltpu.VMEM),
        scratch_shapes=[
            pltpu.VMEM((m, n), x.dtype),
            pltpu.VMEM((m, n), x.dtype),
            pltpu.SemaphoreType.DMA((M,)),
            pltpu.SemaphoreType.DMA((M,)),
            pltpu.SemaphoreType.DMA((M,)),
            pltpu.SemaphoreType.DMA((M,)),
        ],
        compiler_params=pltpu.CompilerParams(collective_id=0),
    )(x)
